# baseline (speedup 1.0000x reference)
import jax
import jax.numpy as jnp
from jax.experimental import pallas as pl
from jax.experimental.pallas import tpu as pltpu

_N_BINS = 10
_BLOCK_ROWS = 1024


def _ce_kernel(probas_ref, labels_ref, lo_ref, hi_ref, out_ref,
               cnt_ref, conf_ref, acc_ref):
    i = pl.program_id(0)
    nsteps = pl.num_programs(0)

    @pl.when(i == 0)
    def _init():
        cnt_ref[...] = jnp.zeros_like(cnt_ref)
        conf_ref[...] = jnp.zeros_like(conf_ref)
        acc_ref[...] = jnp.zeros_like(acc_ref)

    x = probas_ref[...]
    r, c = x.shape
    m = jnp.max(x, axis=-1, keepdims=True)
    iota = jax.lax.broadcasted_iota(jnp.int32, (r, c), 1)
    idx = jnp.min(jnp.where(x == m, iota, c), axis=-1, keepdims=True)
    correct = (idx == labels_ref[...]).astype(jnp.float32)

    lo = lo_ref[...]
    hi = hi_ref[...]
    in_bin = ((lo < m) & (m <= hi)).astype(jnp.float32)
    cnt_ref[...] += jnp.sum(in_bin, axis=0, keepdims=True)
    conf_ref[...] += jnp.sum(in_bin * m, axis=0, keepdims=True)
    acc_ref[...] += jnp.sum(in_bin * correct, axis=0, keepdims=True)

    @pl.when(i == nsteps - 1)
    def _finish():
        cnt = cnt_ref[...]
        total = jnp.sum(cnt)
        valid = (cnt > 0).astype(jnp.float32)
        denom = jnp.maximum(cnt, 1.0)
        confs = conf_ref[...] / denom
        accs = acc_ref[...] / denom
        terms = (cnt / total) * (confs - accs) ** 2 * valid
        out_ref[...] = jnp.sum(terms, axis=1, keepdims=True) ** 0.5


def kernel(probas, labels):
    n, c = probas.shape
    r = min(_BLOCK_ROWS, n)
    grid = n // r

    bins = jnp.linspace(0.0, 1.0, _N_BINS + 1)
    lo = bins[:-1].reshape(1, _N_BINS)
    hi = bins[1:].reshape(1, _N_BINS)
    labels2d = labels.reshape(n, 1)

    out = pl.pallas_call(
        _ce_kernel,
        grid=(grid,),
        in_specs=[
            pl.BlockSpec((r, c), lambda i: (i, 0)),
            pl.BlockSpec((r, 1), lambda i: (i, 0)),
            pl.BlockSpec((1, _N_BINS), lambda i: (0, 0)),
            pl.BlockSpec((1, _N_BINS), lambda i: (0, 0)),
        ],
        out_specs=pl.BlockSpec((1, 1), lambda i: (0, 0)),
        out_shape=jax.ShapeDtypeStruct((1, 1), jnp.float32),
        scratch_shapes=[
            pltpu.VMEM((1, _N_BINS), jnp.float32),
            pltpu.VMEM((1, _N_BINS), jnp.float32),
            pltpu.VMEM((1, _N_BINS), jnp.float32),
        ],
        compiler_params=pltpu.CompilerParams(
            dimension_semantics=("arbitrary",),
        ),
    )(probas, labels2d, lo, hi)
    return out[0, 0]
